# Initial kernel scaffold; baseline (speedup 1.0000x reference)
import functools

import jax
import jax.numpy as jnp
from jax import lax
from jax.experimental import pallas as pl
from jax.experimental.pallas import tpu as pltpu
from jax.experimental.pallas import tpu_sc as plsc

N = 10000
E = 320000
D = 128

NC = 2
NS = 16
L = 16
NW = NC * NS
EPW = E // NW
K = 80
NCHUNK = EPW // K
ROWS_PER_TILE = N // NS

_mesh = plsc.VectorSubcoreMesh(core_axis_name="c", subcore_axis_name="s")


def _w16_body(w_ref, o_ref):
    o_ref[...] = jnp.broadcast_to(w_ref[...], (w_ref.shape[0], L))


def _w16(w):
    return pl.pallas_call(
        _w16_body,
        grid=(40,),
        in_specs=[pl.BlockSpec((E // 40, 1), lambda i: (i, 0))],
        out_specs=pl.BlockSpec((E // 40, L), lambda i: (i, 0)),
        out_shape=jax.ShapeDtypeStruct((E, L), jnp.float32),
    )(w.reshape(E, 1))


def _lin1_body(x_ref, w1_ref, degp_ref, y_ref, dis_ref):
    deg = 1.0 + degp_ref[0, :, 0:1] + degp_ref[1, :, 0:1]
    dis = lax.rsqrt(deg)
    y_ref[...] = jnp.dot(x_ref[...], w1_ref[...],
                         preferred_element_type=jnp.float32) * dis
    dis_ref[...] = dis


def _lin1(x, W1, degp):
    B = 1000
    return pl.pallas_call(
        _lin1_body,
        grid=(N // B,),
        in_specs=[
            pl.BlockSpec((B, D), lambda i: (i, 0)),
            pl.BlockSpec((D, D), lambda i: (0, 0)),
            pl.BlockSpec((NC, B, L), lambda i: (0, i, 0)),
        ],
        out_specs=[
            pl.BlockSpec((B, D), lambda i: (i, 0)),
            pl.BlockSpec((B, 1), lambda i: (i, 0)),
        ],
        out_shape=[
            jax.ShapeDtypeStruct((N, D), jnp.float32),
            jax.ShapeDtypeStruct((N, 1), jnp.float32),
        ],
    )(x, W1, degp)


def _mid_body(p_ref, y1_ref, dis_ref, y2_ref):
    dis = dis_ref[...]
    t = dis * (p_ref[0] + p_ref[1] + y1_ref[...])
    y2_ref[...] = dis * jnp.maximum(t, 0.0)


def _mid(p, y1, dis):
    B = 1000
    return pl.pallas_call(
        _mid_body,
        grid=(N // B,),
        in_specs=[
            pl.BlockSpec((NC, B, D), lambda i: (0, i, 0)),
            pl.BlockSpec((B, D), lambda i: (i, 0)),
            pl.BlockSpec((B, 1), lambda i: (i, 0)),
        ],
        out_specs=pl.BlockSpec((B, D), lambda i: (i, 0)),
        out_shape=jax.ShapeDtypeStruct((N, D), jnp.float32),
    )(p, y1, dis)


def _fin_body(q_ref, y2_ref, dis_ref, w2_ref, o_ref):
    agg = dis_ref[...] * (q_ref[0] + q_ref[1] + y2_ref[...])
    o_ref[...] = jnp.dot(agg, w2_ref[...], preferred_element_type=jnp.float32)


def _fin(q, y2, dis, W2):
    B = 1000
    return pl.pallas_call(
        _fin_body,
        grid=(N // B,),
        in_specs=[
            pl.BlockSpec((NC, B, D), lambda i: (0, i, 0)),
            pl.BlockSpec((B, D), lambda i: (i, 0)),
            pl.BlockSpec((B, 1), lambda i: (i, 0)),
            pl.BlockSpec((D, D), lambda i: (0, 0)),
        ],
        out_specs=pl.BlockSpec((B, D), lambda i: (i, 0)),
        out_shape=jax.ShapeDtypeStruct((N, D), jnp.float32),
    )(q, y2, dis, W2)


@functools.partial(
    pl.kernel,
    mesh=_mesh,
    out_type=jax.ShapeDtypeStruct((NC, N, L), jnp.float32),
    scratch_types=[
        pltpu.VMEM((K,), jnp.int32),
        pltpu.VMEM((K, L), jnp.float32),
        pltpu.VMEM_SHARED((N, L), jnp.float32),
    ],
)
def _deg_kernel(w16_hbm, col_hbm, z_hbm, out_hbm, colv, wbuf, acc):
    cid = lax.axis_index("c")
    sid = lax.axis_index("s")
    wid = cid * NS + sid
    r0 = sid * ROWS_PER_TILE
    pltpu.sync_copy(z_hbm.at[pl.ds(r0, ROWS_PER_TILE)],
                    acc.at[pl.ds(r0, ROWS_PER_TILE)])
    plsc.subcore_barrier()

    @pl.loop(0, NCHUNK)
    def _chunk(ci):
        off = wid * EPW + ci * K
        pltpu.sync_copy(col_hbm.at[pl.ds(off, K)], colv)
        pltpu.sync_copy(w16_hbm.at[pl.ds(off, K)], wbuf)
        pltpu.sync_copy(wbuf, acc.at[colv], add=True)

    plsc.subcore_barrier()
    pltpu.sync_copy(acc.at[pl.ds(r0, ROWS_PER_TILE)],
                    out_hbm.at[cid, pl.ds(r0, ROWS_PER_TILE)])


@functools.partial(
    pl.kernel,
    mesh=_mesh,
    out_type=jax.ShapeDtypeStruct((NC, N, D), jnp.float32),
    scratch_types=[
        pltpu.VMEM((K,), jnp.int32),
        pltpu.VMEM((K,), jnp.int32),
        pltpu.VMEM((K, L), jnp.float32),
        pltpu.VMEM((K, D), jnp.float32),
        pltpu.VMEM_SHARED((N, D), jnp.float32),
        pltpu.SemaphoreType.DMA,
    ],
)
def _agg_kernel(y_hbm, row_hbm, col_hbm, w16_hbm, z_hbm, out_hbm,
                rowv, colv, wbuf, rows, acc, sem):
    cid = lax.axis_index("c")
    sid = lax.axis_index("s")
    wid = cid * NS + sid
    r0 = sid * ROWS_PER_TILE
    pltpu.sync_copy(z_hbm.at[pl.ds(r0, ROWS_PER_TILE)],
                    acc.at[pl.ds(r0, ROWS_PER_TILE)])
    plsc.subcore_barrier()

    @pl.loop(0, NCHUNK)
    def _chunk(ci):
        off = wid * EPW + ci * K
        pltpu.sync_copy(row_hbm.at[pl.ds(off, K)], rowv)
        pltpu.sync_copy(col_hbm.at[pl.ds(off, K)], colv)
        pltpu.sync_copy(w16_hbm.at[pl.ds(off, K)], wbuf)
        pltpu.async_copy(y_hbm.at[rowv], rows, sem).wait()

        @pl.loop(0, K)
        def _edge(e):
            s = wbuf[e, :]
            for c in range(D // L):
                sl = (e, pl.ds(c * L, L))
                rows[sl] = rows[sl] * s

        pltpu.sync_copy(rows, acc.at[colv], add=True)

    plsc.subcore_barrier()
    pltpu.sync_copy(acc.at[pl.ds(r0, ROWS_PER_TILE)],
                    out_hbm.at[cid, pl.ds(r0, ROWS_PER_TILE)])


def kernel(x, edge_index, edge_weight, W1, W2):
    row = edge_index[0].astype(jnp.int32)
    col = edge_index[1].astype(jnp.int32)
    w = edge_weight.astype(jnp.float32)

    w16 = _w16(w)
    zL = jnp.zeros((N, L), jnp.float32)
    zD = jnp.zeros((N, D), jnp.float32)

    degp = _deg_kernel(w16, col, zL)
    y1, dis = _lin1(x, W1, degp)
    p = _agg_kernel(y1, row, col, w16, zD)
    y2 = _mid(p, y1, dis)
    q = _agg_kernel(y2, row, col, w16, zD)
    return _fin(q, y2, dis, W2)


# speedup vs baseline: 7.7302x; 7.7302x over previous
import dataclasses
import functools

import numpy as np

import jax
import jax.numpy as jnp
from jax import lax
from jax.experimental import pallas as pl
from jax.experimental.pallas import tpu as pltpu
from jax.experimental.pallas import tpu_sc as plsc

N = 10000
E = 320000
D = 128

NC = 2
NS = 16
L = 16
NW = NC * NS
EPW = E // NW
K = 80
NCHUNK = EPW // K
NP = 10240
ROWS_PER_TILE = NP // NS

_mesh = plsc.VectorSubcoreMesh(core_axis_name="c", subcore_axis_name="s")

_sc_params = pltpu.CompilerParams()
if "needs_layout_passes" in pltpu.CompilerParams.__dataclass_fields__:
    _sc_params = dataclasses.replace(_sc_params, needs_layout_passes=False)


def _w16_body(w_ref, o_ref):
    o_ref[...] = jnp.broadcast_to(w_ref[...], (w_ref.shape[0], L))


def _w16(w):
    return pl.pallas_call(
        _w16_body,
        grid=(40,),
        in_specs=[pl.BlockSpec((E // 40, 1), lambda i: (i, 0))],
        out_specs=pl.BlockSpec((E // 40, L), lambda i: (i, 0)),
        out_shape=jax.ShapeDtypeStruct((E, L), jnp.float32),
    )(w.reshape(E, 1))


def _lin1_body(x_ref, w1_ref, degp_ref, y_ref, dis_ref):
    deg = 1.0 + jnp.sum(degp_ref[...], axis=0)[:, None]
    dis = lax.rsqrt(deg)
    y_ref[...] = jnp.dot(x_ref[...], w1_ref[...],
                         preferred_element_type=jnp.float32) * dis
    dis_ref[...] = dis


def _lin1(x, W1, degp):
    B = 1280
    return pl.pallas_call(
        _lin1_body,
        grid=(NP // B,),
        in_specs=[
            pl.BlockSpec((B, D), lambda i: (i, 0)),
            pl.BlockSpec((D, D), lambda i: (0, 0)),
            pl.BlockSpec((NW, B), lambda i: (0, i)),
        ],
        out_specs=[
            pl.BlockSpec((B, D), lambda i: (i, 0)),
            pl.BlockSpec((B, 1), lambda i: (i, 0)),
        ],
        out_shape=[
            jax.ShapeDtypeStruct((NP, D), jnp.float32),
            jax.ShapeDtypeStruct((NP, 1), jnp.float32),
        ],
    )(x, W1, degp)


def _mid_body(p_ref, y1_ref, dis_ref, y2_ref):
    dis = dis_ref[...]
    t = dis * (p_ref[0] + p_ref[1] + y1_ref[...])
    y2_ref[...] = dis * jnp.maximum(t, 0.0)


def _mid(p, y1, dis):
    B = 1280
    return pl.pallas_call(
        _mid_body,
        grid=(NP // B,),
        in_specs=[
            pl.BlockSpec((NC, B, D), lambda i: (0, i, 0)),
            pl.BlockSpec((B, D), lambda i: (i, 0)),
            pl.BlockSpec((B, 1), lambda i: (i, 0)),
        ],
        out_specs=pl.BlockSpec((B, D), lambda i: (i, 0)),
        out_shape=jax.ShapeDtypeStruct((NP, D), jnp.float32),
    )(p, y1, dis)


def _fin_body(q_ref, y2_ref, dis_ref, w2_ref, o_ref):
    agg = dis_ref[...] * (q_ref[0] + q_ref[1] + y2_ref[...])
    o_ref[...] = jnp.dot(agg, w2_ref[...], preferred_element_type=jnp.float32)


def _fin(q, y2, dis, W2):
    B = 1280
    return pl.pallas_call(
        _fin_body,
        grid=(NP // B,),
        in_specs=[
            pl.BlockSpec((NC, B, D), lambda i: (0, i, 0)),
            pl.BlockSpec((B, D), lambda i: (i, 0)),
            pl.BlockSpec((B, 1), lambda i: (i, 0)),
            pl.BlockSpec((D, D), lambda i: (0, 0)),
        ],
        out_specs=pl.BlockSpec((B, D), lambda i: (i, 0)),
        out_shape=jax.ShapeDtypeStruct((NP, D), jnp.float32),
    )(q, y2, dis, W2)


@functools.partial(
    pl.kernel,
    mesh=_mesh,
    compiler_params=_sc_params,
    out_type=jax.ShapeDtypeStruct((NW, NP), jnp.float32),
    scratch_types=[
        pltpu.VMEM((EPW,), jnp.int32),
        pltpu.VMEM((EPW,), jnp.float32),
        pltpu.VMEM((NP,), jnp.float32),
    ],
)
def _deg_kernel(w_hbm, col_hbm, out_hbm, colv, wv, degv):
    cid = lax.axis_index("c")
    sid = lax.axis_index("s")
    wid = cid * NS + sid
    lanes = lax.iota(jnp.int32, L)
    zero = jnp.zeros((L,), jnp.float32)

    @pl.loop(0, NP, step=L)
    def _z(i):
        degv[pl.ds(i, L)] = zero

    pltpu.sync_copy(col_hbm.at[pl.ds(wid * EPW, EPW)], colv)
    pltpu.sync_copy(w_hbm.at[pl.ds(wid * EPW, EPW)], wv)

    @pl.loop(0, EPW, step=L)
    def _grp(g):
        idx = colv[pl.ds(g, L)]
        vals = wv[pl.ds(g, L)]
        for j in range(L):
            plsc.addupdate_scatter(degv, [idx], vals, mask=lanes == j)

    pltpu.sync_copy(degv, out_hbm.at[wid])


@functools.partial(
    pl.kernel,
    mesh=_mesh,
    out_type=jax.ShapeDtypeStruct((NC, NP, D), jnp.float32),
    scratch_types=[
        pltpu.VMEM((K,), jnp.int32),
        pltpu.VMEM((K,), jnp.int32),
        pltpu.VMEM((K, L), jnp.float32),
        pltpu.VMEM((K, D), jnp.float32),
        pltpu.VMEM_SHARED((NP, D), jnp.float32),
        pltpu.SemaphoreType.DMA,
    ],
)
def _agg_kernel(y_hbm, row_hbm, col_hbm, w16_hbm, z_hbm, out_hbm,
                rowv, colv, wbuf, rows, acc, sem):
    cid = lax.axis_index("c")
    sid = lax.axis_index("s")
    wid = cid * NS + sid
    r0 = sid * ROWS_PER_TILE
    pltpu.sync_copy(z_hbm.at[pl.ds(r0, ROWS_PER_TILE)],
                    acc.at[pl.ds(r0, ROWS_PER_TILE)])
    plsc.subcore_barrier()

    @pl.loop(0, NCHUNK)
    def _chunk(ci):
        off = wid * EPW + ci * K
        pltpu.sync_copy(row_hbm.at[pl.ds(off, K)], rowv)
        pltpu.sync_copy(col_hbm.at[pl.ds(off, K)], colv)
        pltpu.sync_copy(w16_hbm.at[pl.ds(off, K)], wbuf)
        pltpu.async_copy(y_hbm.at[rowv], rows, sem).wait()

        @pl.loop(0, K)
        def _edge(e):
            s = wbuf[e, :]
            for c in range(D // L):
                sl = (e, pl.ds(c * L, L))
                rows[sl] = rows[sl] * s

        pltpu.sync_copy(rows, acc.at[colv], add=True)

    plsc.subcore_barrier()
    pltpu.sync_copy(acc.at[pl.ds(r0, ROWS_PER_TILE)],
                    out_hbm.at[cid, pl.ds(r0, ROWS_PER_TILE)])


def kernel(x, edge_index, edge_weight, W1, W2):
    row = edge_index[0].astype(jnp.int32)
    col = edge_index[1].astype(jnp.int32)
    w = edge_weight.astype(jnp.float32)

    w16 = _w16(w)
    zD = jnp.zeros((NP, D), jnp.float32)
    xp = jnp.pad(x, ((0, NP - N), (0, 0)))

    degp = _deg_kernel(w, col)
    y1, dis = _lin1(xp, W1, degp)
    p = _agg_kernel(y1, row, col, w16, zD)
    y2 = _mid(p, y1, dis)
    q = _agg_kernel(y2, row, col, w16, zD)
    return _fin(q, y2, dis, W2)[:N]


# speedup vs baseline: 9.3988x; 1.2159x over previous
import dataclasses
import functools

import numpy as np

import jax
import jax.numpy as jnp
from jax import lax
from jax.experimental import pallas as pl
from jax.experimental.pallas import tpu as pltpu
from jax.experimental.pallas import tpu_sc as plsc

N = 10000
E = 320000
D = 128

NC = 2
NS = 16
L = 16
NW = NC * NS
EPW = E // NW
K = 32
EPA = 10240
NCHA = EPA // K
EA = EPA * NW
NP = 10240
ROWS_PER_TILE = NP // NS

_mesh = plsc.VectorSubcoreMesh(core_axis_name="c", subcore_axis_name="s")

_sc_params = pltpu.CompilerParams()
if "needs_layout_passes" in pltpu.CompilerParams.__dataclass_fields__:
    _sc_params = dataclasses.replace(_sc_params, needs_layout_passes=False)


def _w16_body(w_ref, o_ref):
    o_ref[...] = jnp.broadcast_to(w_ref[...], (w_ref.shape[0], L))


def _w16(w):
    return pl.pallas_call(
        _w16_body,
        grid=(40,),
        in_specs=[pl.BlockSpec((E // 40, 1), lambda i: (i, 0))],
        out_specs=pl.BlockSpec((E // 40, L), lambda i: (i, 0)),
        out_shape=jax.ShapeDtypeStruct((E, L), jnp.float32),
    )(w.reshape(E, 1))


def _lin1_body(x_ref, w1_ref, degp_ref, y_ref, dis_ref):
    deg = 1.0 + jnp.sum(degp_ref[...], axis=0)[:, None]
    dis = lax.rsqrt(deg)
    y_ref[...] = jnp.dot(x_ref[...], w1_ref[...],
                         preferred_element_type=jnp.float32) * dis
    dis_ref[...] = dis


def _lin1(x, W1, degp):
    B = 1280
    return pl.pallas_call(
        _lin1_body,
        grid=(NP // B,),
        in_specs=[
            pl.BlockSpec((B, D), lambda i: (i, 0)),
            pl.BlockSpec((D, D), lambda i: (0, 0)),
            pl.BlockSpec((NW, B), lambda i: (0, i)),
        ],
        out_specs=[
            pl.BlockSpec((B, D), lambda i: (i, 0)),
            pl.BlockSpec((B, 1), lambda i: (i, 0)),
        ],
        out_shape=[
            jax.ShapeDtypeStruct((NP, D), jnp.float32),
            jax.ShapeDtypeStruct((NP, 1), jnp.float32),
        ],
    )(x, W1, degp)


def _mid_body(p_ref, y1_ref, dis_ref, y2_ref):
    dis = dis_ref[...]
    t = dis * (p_ref[0] + p_ref[1] + y1_ref[...])
    y2_ref[...] = dis * jnp.maximum(t, 0.0)


def _mid(p, y1, dis):
    B = 1280
    return pl.pallas_call(
        _mid_body,
        grid=(NP // B,),
        in_specs=[
            pl.BlockSpec((NC, B, D), lambda i: (0, i, 0)),
            pl.BlockSpec((B, D), lambda i: (i, 0)),
            pl.BlockSpec((B, 1), lambda i: (i, 0)),
        ],
        out_specs=pl.BlockSpec((B, D), lambda i: (i, 0)),
        out_shape=jax.ShapeDtypeStruct((NP, D), jnp.float32),
    )(p, y1, dis)


def _fin_body(q_ref, y2_ref, dis_ref, w2_ref, o_ref):
    agg = dis_ref[...] * (q_ref[0] + q_ref[1] + y2_ref[...])
    o_ref[...] = jnp.dot(agg, w2_ref[...], preferred_element_type=jnp.float32)


def _fin(q, y2, dis, W2):
    B = 1280
    return pl.pallas_call(
        _fin_body,
        grid=(NP // B,),
        in_specs=[
            pl.BlockSpec((NC, B, D), lambda i: (0, i, 0)),
            pl.BlockSpec((B, D), lambda i: (i, 0)),
            pl.BlockSpec((B, 1), lambda i: (i, 0)),
            pl.BlockSpec((D, D), lambda i: (0, 0)),
        ],
        out_specs=pl.BlockSpec((B, D), lambda i: (i, 0)),
        out_shape=jax.ShapeDtypeStruct((NP, D), jnp.float32),
    )(q, y2, dis, W2)


@functools.partial(
    pl.kernel,
    mesh=_mesh,
    compiler_params=_sc_params,
    out_type=jax.ShapeDtypeStruct((NW, NP), jnp.float32),
    scratch_types=[
        pltpu.VMEM((EPW,), jnp.int32),
        pltpu.VMEM((EPW,), jnp.float32),
        pltpu.VMEM((NP,), jnp.float32),
    ],
)
def _deg_kernel(w_hbm, col_hbm, out_hbm, colv, wv, degv):
    cid = lax.axis_index("c")
    sid = lax.axis_index("s")
    wid = cid * NS + sid
    lanes = lax.iota(jnp.int32, L)
    zero = jnp.zeros((L,), jnp.float32)

    @pl.loop(0, NP, step=L)
    def _z(i):
        degv[pl.ds(i, L)] = zero

    pltpu.sync_copy(col_hbm.at[pl.ds(wid * EPW, EPW)], colv)
    pltpu.sync_copy(w_hbm.at[pl.ds(wid * EPW, EPW)], wv)

    @pl.loop(0, EPW, step=L)
    def _grp(g):
        idx = colv[pl.ds(g, L)]
        vals = wv[pl.ds(g, L)]
        for j in range(L):
            plsc.addupdate_scatter(degv, [idx], vals, mask=lanes == j)

    pltpu.sync_copy(degv, out_hbm.at[wid])


@functools.partial(
    pl.kernel,
    mesh=_mesh,
    out_type=jax.ShapeDtypeStruct((NC, NP, D), jnp.float32),
    scratch_types=[
        pltpu.VMEM((EPA,), jnp.int32),
        pltpu.VMEM((EPA,), jnp.int32),
        pltpu.VMEM((EPA,), jnp.float32),
        pltpu.VMEM((K, D), jnp.float32),
        pltpu.VMEM((K, D), jnp.float32),
        pltpu.VMEM_SHARED((NP, D), jnp.float32),
        pltpu.SemaphoreType.DMA,
        pltpu.SemaphoreType.DMA,
    ],
)
def _agg_kernel(y_hbm, row_hbm, col_hbm, w_hbm, z_hbm, out_hbm,
                rowv, colv, wv, rows0, rows1, acc, sem0, sem1):
    cid = lax.axis_index("c")
    sid = lax.axis_index("s")
    wid = cid * NS + sid
    r0 = sid * ROWS_PER_TILE
    pltpu.sync_copy(row_hbm.at[pl.ds(wid * EPA, EPA)], rowv)
    pltpu.sync_copy(col_hbm.at[pl.ds(wid * EPA, EPA)], colv)
    pltpu.sync_copy(w_hbm.at[pl.ds(wid * EPA, EPA)], wv)
    pltpu.sync_copy(z_hbm.at[pl.ds(r0, ROWS_PER_TILE)],
                    acc.at[pl.ds(r0, ROWS_PER_TILE)])
    plsc.subcore_barrier()

    def _gather(c, buf, sem):
        return pltpu.async_copy(y_hbm.at[rowv.at[pl.ds(c * K, K)]], buf, sem)

    def _scale(buf, c):
        @pl.loop(0, K, step=L)
        def _grp(g):
            wreg = wv[pl.ds(c * K + g, L)]
            for j in range(L):
                sp = lax.gather(
                    wreg, jnp.full((L, 1), j, jnp.int32),
                    lax.GatherDimensionNumbers(
                        offset_dims=(), collapsed_slice_dims=(0,),
                        start_index_map=(0,)),
                    slice_sizes=(1,),
                    mode=lax.GatherScatterMode.PROMISE_IN_BOUNDS)
                for cdx in range(D // L):
                    sl = (g + j, pl.ds(cdx * L, L))
                    buf[sl] = buf[sl] * sp

    def _scatter(buf, c):
        pltpu.sync_copy(buf, acc.at[colv.at[pl.ds(c * K, K)]], add=True)

    def _wait(c, buf, sem):
        pltpu.make_async_copy(
            y_hbm.at[rowv.at[pl.ds(c * K, K)]], buf, sem).wait()

    _gather(0, rows0, sem0)

    @pl.loop(0, NCHA, step=2)
    def _main(ci):
        _gather(ci + 1, rows1, sem1)
        _wait(ci, rows0, sem0)
        _scale(rows0, ci)
        _scatter(rows0, ci)

        @pl.when(ci + 2 < NCHA)
        def _pre():
            _gather(ci + 2, rows0, sem0)

        _wait(ci + 1, rows1, sem1)
        _scale(rows1, ci + 1)
        _scatter(rows1, ci + 1)

    plsc.subcore_barrier()
    pltpu.sync_copy(acc.at[pl.ds(r0, ROWS_PER_TILE)],
                    out_hbm.at[cid, pl.ds(r0, ROWS_PER_TILE)])


def kernel(x, edge_index, edge_weight, W1, W2):
    row = edge_index[0].astype(jnp.int32)
    col = edge_index[1].astype(jnp.int32)
    w = edge_weight.astype(jnp.float32)

    rowp = jnp.pad(row.reshape(NW, EPW), ((0, 0), (0, EPA - EPW))).reshape(EA)
    colp = jnp.pad(col.reshape(NW, EPW), ((0, 0), (0, EPA - EPW))).reshape(EA)
    wp = jnp.pad(w.reshape(NW, EPW), ((0, 0), (0, EPA - EPW))).reshape(EA)
    zD = jnp.zeros((NP, D), jnp.float32)
    xp = jnp.pad(x, ((0, NP - N), (0, 0)))

    degp = _deg_kernel(w, col)
    y1, dis = _lin1(xp, W1, degp)
    p = _agg_kernel(y1, rowp, colp, wp, zD)
    y2 = _mid(p, y1, dis)
    q = _agg_kernel(y2, rowp, colp, wp, zD)
    return _fin(q, y2, dis, W2)[:N]


# speedup vs baseline: 9.9352x; 1.0571x over previous
import dataclasses
import functools

import numpy as np

import jax
import jax.numpy as jnp
from jax import lax
from jax.experimental import pallas as pl
from jax.experimental.pallas import tpu as pltpu
from jax.experimental.pallas import tpu_sc as plsc

N = 10000
E = 320000
D = 128

NC = 2
NS = 16
L = 16
NW = NC * NS
EPW = E // NW
K = 32
EPA = 10240
NCHA = EPA // K
EA = EPA * NW
NP = 10240
ROWS_PER_TILE = NP // NS

_mesh = plsc.VectorSubcoreMesh(core_axis_name="c", subcore_axis_name="s")

_sc_params = pltpu.CompilerParams()
if "needs_layout_passes" in pltpu.CompilerParams.__dataclass_fields__:
    _sc_params = dataclasses.replace(_sc_params, needs_layout_passes=False)


def _w16_body(w_ref, o_ref):
    o_ref[...] = jnp.broadcast_to(w_ref[...], (w_ref.shape[0], L))


def _w16(w):
    return pl.pallas_call(
        _w16_body,
        grid=(40,),
        in_specs=[pl.BlockSpec((E // 40, 1), lambda i: (i, 0))],
        out_specs=pl.BlockSpec((E // 40, L), lambda i: (i, 0)),
        out_shape=jax.ShapeDtypeStruct((E, L), jnp.float32),
    )(w.reshape(E, 1))


def _lin1_body(x_ref, w1_ref, degp_ref, y_ref, dis_ref):
    deg = 1.0 + jnp.sum(degp_ref[...], axis=0)[:, None]
    dis = lax.rsqrt(deg)
    y_ref[...] = jnp.dot(x_ref[...], w1_ref[...],
                         preferred_element_type=jnp.float32) * dis
    dis_ref[...] = dis


def _lin1(x, W1, degp):
    B = 1280
    return pl.pallas_call(
        _lin1_body,
        grid=(NP // B,),
        in_specs=[
            pl.BlockSpec((B, D), lambda i: (i, 0)),
            pl.BlockSpec((D, D), lambda i: (0, 0)),
            pl.BlockSpec((NW, B), lambda i: (0, i)),
        ],
        out_specs=[
            pl.BlockSpec((B, D), lambda i: (i, 0)),
            pl.BlockSpec((B, 1), lambda i: (i, 0)),
        ],
        out_shape=[
            jax.ShapeDtypeStruct((NP, D), jnp.float32),
            jax.ShapeDtypeStruct((NP, 1), jnp.float32),
        ],
    )(x, W1, degp)


def _mid_body(p_ref, y1_ref, dis_ref, y2_ref):
    dis = dis_ref[...]
    t = dis * (p_ref[0] + p_ref[1] + y1_ref[...])
    y2_ref[...] = dis * jnp.maximum(t, 0.0)


def _mid(p, y1, dis):
    B = 1280
    return pl.pallas_call(
        _mid_body,
        grid=(NP // B,),
        in_specs=[
            pl.BlockSpec((NC, B, D), lambda i: (0, i, 0)),
            pl.BlockSpec((B, D), lambda i: (i, 0)),
            pl.BlockSpec((B, 1), lambda i: (i, 0)),
        ],
        out_specs=pl.BlockSpec((B, D), lambda i: (i, 0)),
        out_shape=jax.ShapeDtypeStruct((NP, D), jnp.float32),
    )(p, y1, dis)


def _fin_body(q_ref, y2_ref, dis_ref, w2_ref, o_ref):
    agg = dis_ref[...] * (q_ref[0] + q_ref[1] + y2_ref[...])
    o_ref[...] = jnp.dot(agg, w2_ref[...], preferred_element_type=jnp.float32)


def _fin(q, y2, dis, W2):
    B = 1280
    return pl.pallas_call(
        _fin_body,
        grid=(NP // B,),
        in_specs=[
            pl.BlockSpec((NC, B, D), lambda i: (0, i, 0)),
            pl.BlockSpec((B, D), lambda i: (i, 0)),
            pl.BlockSpec((B, 1), lambda i: (i, 0)),
            pl.BlockSpec((D, D), lambda i: (0, 0)),
        ],
        out_specs=pl.BlockSpec((B, D), lambda i: (i, 0)),
        out_shape=jax.ShapeDtypeStruct((NP, D), jnp.float32),
    )(q, y2, dis, W2)


@functools.partial(
    pl.kernel,
    mesh=_mesh,
    compiler_params=_sc_params,
    out_type=jax.ShapeDtypeStruct((NW, NP), jnp.float32),
    scratch_types=[
        pltpu.VMEM((EPW,), jnp.int32),
        pltpu.VMEM((EPW,), jnp.float32),
        pltpu.VMEM((NP,), jnp.float32),
    ],
)
def _deg_kernel(w_hbm, col_hbm, out_hbm, colv, wv, degv):
    cid = lax.axis_index("c")
    sid = lax.axis_index("s")
    wid = cid * NS + sid
    lanes = lax.iota(jnp.int32, L)
    zero = jnp.zeros((L,), jnp.float32)

    @pl.loop(0, NP, step=L)
    def _z(i):
        degv[pl.ds(i, L)] = zero

    pltpu.sync_copy(col_hbm.at[pl.ds(wid * EPW, EPW)], colv)
    pltpu.sync_copy(w_hbm.at[pl.ds(wid * EPW, EPW)], wv)

    @pl.loop(0, EPW, step=L)
    def _grp(g):
        idx = colv[pl.ds(g, L)]
        vals = wv[pl.ds(g, L)]
        for j in range(L):
            plsc.addupdate_scatter(degv, [idx], vals, mask=lanes == j)

    pltpu.sync_copy(degv, out_hbm.at[wid])


@functools.partial(
    pl.kernel,
    mesh=_mesh,
    out_type=jax.ShapeDtypeStruct((NC, NP, D), jnp.float32),
    scratch_types=[
        pltpu.VMEM((EPA,), jnp.int32),
        pltpu.VMEM((EPA,), jnp.int32),
        pltpu.VMEM((EPA,), jnp.float32),
        pltpu.VMEM((K, D), jnp.float32),
        pltpu.VMEM((K, D), jnp.float32),
        pltpu.VMEM((K, D), jnp.float32),
        pltpu.VMEM((K, D), jnp.float32),
        pltpu.VMEM_SHARED((NP, D), jnp.float32),
        pltpu.SemaphoreType.DMA,
        pltpu.SemaphoreType.DMA,
        pltpu.SemaphoreType.DMA,
        pltpu.SemaphoreType.DMA,
        pltpu.SemaphoreType.DMA,
        pltpu.SemaphoreType.DMA,
        pltpu.SemaphoreType.DMA,
        pltpu.SemaphoreType.DMA,
    ],
)
def _agg_kernel(y_hbm, row_hbm, col_hbm, w_hbm, z_hbm, out_hbm,
                rowv, colv, wv, rb0, rb1, rb2, rb3, acc,
                g0, g1, g2, g3, s0, s1, s2, s3):
    rows = [rb0, rb1, rb2, rb3]
    gsem = [g0, g1, g2, g3]
    ssem = [s0, s1, s2, s3]
    cid = lax.axis_index("c")
    sid = lax.axis_index("s")
    wid = cid * NS + sid
    r0 = sid * ROWS_PER_TILE
    pltpu.sync_copy(row_hbm.at[pl.ds(wid * EPA, EPA)], rowv)
    pltpu.sync_copy(col_hbm.at[pl.ds(wid * EPA, EPA)], colv)
    pltpu.sync_copy(w_hbm.at[pl.ds(wid * EPA, EPA)], wv)
    pltpu.sync_copy(z_hbm.at[pl.ds(r0, ROWS_PER_TILE)],
                    acc.at[pl.ds(r0, ROWS_PER_TILE)])
    plsc.subcore_barrier()

    def _gather(c, b):
        pltpu.async_copy(y_hbm.at[rowv.at[pl.ds(c * K, K)]], rows[b], gsem[b])

    def _wait_g(c, b):
        pltpu.make_async_copy(
            y_hbm.at[rowv.at[pl.ds(c * K, K)]], rows[b], gsem[b]).wait()

    def _scatter(c, b):
        pltpu.async_copy(rows[b], acc.at[colv.at[pl.ds(c * K, K)]], ssem[b],
                         add=True)

    def _wait_s(c, b):
        pltpu.make_async_copy(
            rows[b], acc.at[colv.at[pl.ds(c * K, K)]], ssem[b]).wait()

    def _scale(c, b):
        buf = rows[b]

        @pl.loop(0, K, step=L)
        def _grp(g):
            wreg = wv[pl.ds(c * K + g, L)]
            for j in range(L):
                sp = lax.gather(
                    wreg, jnp.full((L, 1), j, jnp.int32),
                    lax.GatherDimensionNumbers(
                        offset_dims=(), collapsed_slice_dims=(0,),
                        start_index_map=(0,)),
                    slice_sizes=(1,),
                    mode=lax.GatherScatterMode.PROMISE_IN_BOUNDS)
                for cdx in range(D // L):
                    sl = (g + j, pl.ds(cdx * L, L))
                    buf[sl] = buf[sl] * sp

    _gather(0, 0)
    _gather(1, 1)
    _wait_g(0, 0); _scale(0, 0); _scatter(0, 0); _gather(2, 2)
    _wait_g(1, 1); _scale(1, 1); _scatter(1, 1); _gather(3, 3)

    @pl.loop(2, NCHA - 2, step=4)
    def _main(ci):
        for u in range(4):
            c = ci + u
            b = (2 + u) % 4
            bn = (b + 2) % 4
            _wait_g(c, b)
            _scale(c, b)
            _scatter(c, b)
            _wait_s(c - 2, bn)
            _gather(c + 2, bn)

    _wait_g(NCHA - 2, 2); _scale(NCHA - 2, 2); _scatter(NCHA - 2, 2)
    _wait_g(NCHA - 1, 3); _scale(NCHA - 1, 3); _scatter(NCHA - 1, 3)
    _wait_s(NCHA - 4, 0)
    _wait_s(NCHA - 3, 1)
    _wait_s(NCHA - 2, 2)
    _wait_s(NCHA - 1, 3)

    plsc.subcore_barrier()
    pltpu.sync_copy(acc.at[pl.ds(r0, ROWS_PER_TILE)],
                    out_hbm.at[cid, pl.ds(r0, ROWS_PER_TILE)])


def kernel(x, edge_index, edge_weight, W1, W2):
    row = edge_index[0].astype(jnp.int32)
    col = edge_index[1].astype(jnp.int32)
    w = edge_weight.astype(jnp.float32)

    rowp = jnp.pad(row.reshape(NW, EPW), ((0, 0), (0, EPA - EPW))).reshape(EA)
    colp = jnp.pad(col.reshape(NW, EPW), ((0, 0), (0, EPA - EPW))).reshape(EA)
    wp = jnp.pad(w.reshape(NW, EPW), ((0, 0), (0, EPA - EPW))).reshape(EA)
    zD = jnp.zeros((NP, D), jnp.float32)
    xp = jnp.pad(x, ((0, NP - N), (0, 0)))

    degp = _deg_kernel(w, col)
    y1, dis = _lin1(xp, W1, degp)
    p = _agg_kernel(y1, rowp, colp, wp, zD)
    y2 = _mid(p, y1, dis)
    q = _agg_kernel(y2, rowp, colp, wp, zD)
    return _fin(q, y2, dis, W2)[:N]


# speedup vs baseline: 10.9075x; 1.0979x over previous
import dataclasses
import functools

import numpy as np

import jax
import jax.numpy as jnp
from jax import lax
from jax.experimental import pallas as pl
from jax.experimental.pallas import tpu as pltpu
from jax.experimental.pallas import tpu_sc as plsc

N = 10000
E = 320000
D = 128

NC = 2
NS = 16
L = 16
NW = NC * NS
EPW = E // NW
K = 80
EPA = 10240
NCHA = EPA // K
EA = EPA * NW
NRB = 3
NIB = 6
NP = 10240
ROWS_PER_TILE = NP // NS

_mesh = plsc.VectorSubcoreMesh(core_axis_name="c", subcore_axis_name="s")

_sc_params = pltpu.CompilerParams()
if "needs_layout_passes" in pltpu.CompilerParams.__dataclass_fields__:
    _sc_params = dataclasses.replace(_sc_params, needs_layout_passes=False)


def _w16_body(w_ref, o_ref):
    o_ref[...] = jnp.broadcast_to(w_ref[...], (w_ref.shape[0], L))


def _w16(w):
    return pl.pallas_call(
        _w16_body,
        grid=(40,),
        in_specs=[pl.BlockSpec((E // 40, 1), lambda i: (i, 0))],
        out_specs=pl.BlockSpec((E // 40, L), lambda i: (i, 0)),
        out_shape=jax.ShapeDtypeStruct((E, L), jnp.float32),
    )(w.reshape(E, 1))


def _lin1_body(x_ref, w1_ref, degp_ref, y_ref, dis_ref):
    deg = 1.0 + jnp.sum(degp_ref[...], axis=0)[:, None]
    dis = lax.rsqrt(deg)
    y_ref[...] = jnp.dot(x_ref[...], w1_ref[...],
                         preferred_element_type=jnp.float32) * dis
    dis_ref[...] = dis


def _lin1(x, W1, degp):
    B = 1280
    return pl.pallas_call(
        _lin1_body,
        grid=(NP // B,),
        in_specs=[
            pl.BlockSpec((B, D), lambda i: (i, 0)),
            pl.BlockSpec((D, D), lambda i: (0, 0)),
            pl.BlockSpec((NW, B), lambda i: (0, i)),
        ],
        out_specs=[
            pl.BlockSpec((B, D), lambda i: (i, 0)),
            pl.BlockSpec((B, 1), lambda i: (i, 0)),
        ],
        out_shape=[
            jax.ShapeDtypeStruct((NP, D), jnp.float32),
            jax.ShapeDtypeStruct((NP, 1), jnp.float32),
        ],
    )(x, W1, degp)


def _mid_body(p_ref, y1_ref, dis_ref, y2_ref):
    dis = dis_ref[...]
    t = dis * (p_ref[0] + p_ref[1] + y1_ref[...])
    y2_ref[...] = dis * jnp.maximum(t, 0.0)


def _mid(p, y1, dis):
    B = 1280
    return pl.pallas_call(
        _mid_body,
        grid=(NP // B,),
        in_specs=[
            pl.BlockSpec((NC, B, D), lambda i: (0, i, 0)),
            pl.BlockSpec((B, D), lambda i: (i, 0)),
            pl.BlockSpec((B, 1), lambda i: (i, 0)),
        ],
        out_specs=pl.BlockSpec((B, D), lambda i: (i, 0)),
        out_shape=jax.ShapeDtypeStruct((NP, D), jnp.float32),
    )(p, y1, dis)


def _fin_body(q_ref, y2_ref, dis_ref, w2_ref, o_ref):
    agg = dis_ref[...] * (q_ref[0] + q_ref[1] + y2_ref[...])
    o_ref[...] = jnp.dot(agg, w2_ref[...], preferred_element_type=jnp.float32)


def _fin(q, y2, dis, W2):
    B = 1280
    return pl.pallas_call(
        _fin_body,
        grid=(NP // B,),
        in_specs=[
            pl.BlockSpec((NC, B, D), lambda i: (0, i, 0)),
            pl.BlockSpec((B, D), lambda i: (i, 0)),
            pl.BlockSpec((B, 1), lambda i: (i, 0)),
            pl.BlockSpec((D, D), lambda i: (0, 0)),
        ],
        out_specs=pl.BlockSpec((B, D), lambda i: (i, 0)),
        out_shape=jax.ShapeDtypeStruct((NP, D), jnp.float32),
    )(q, y2, dis, W2)


@functools.partial(
    pl.kernel,
    mesh=_mesh,
    compiler_params=_sc_params,
    out_type=jax.ShapeDtypeStruct((NW, NP), jnp.float32),
    scratch_types=[
        pltpu.VMEM((EPW,), jnp.int32),
        pltpu.VMEM((EPW,), jnp.float32),
        pltpu.VMEM((NP,), jnp.float32),
    ],
)
def _deg_kernel(w_hbm, col_hbm, out_hbm, colv, wv, degv):
    cid = lax.axis_index("c")
    sid = lax.axis_index("s")
    wid = cid * NS + sid
    lanes = lax.iota(jnp.int32, L)
    zero = jnp.zeros((L,), jnp.float32)

    @pl.loop(0, NP, step=L)
    def _z(i):
        degv[pl.ds(i, L)] = zero

    pltpu.sync_copy(col_hbm.at[pl.ds(wid * EPW, EPW)], colv)
    pltpu.sync_copy(w_hbm.at[pl.ds(wid * EPW, EPW)], wv)

    @pl.loop(0, EPW, step=L)
    def _grp(g):
        idx = colv[pl.ds(g, L)]
        vals = wv[pl.ds(g, L)]
        for j in range(L):
            plsc.addupdate_scatter(degv, [idx], vals, mask=lanes == j)

    pltpu.sync_copy(degv, out_hbm.at[wid])


@functools.partial(
    pl.kernel,
    mesh=_mesh,
    out_type=jax.ShapeDtypeStruct((NC, NP, D), jnp.float32),
    scratch_types=(
        [pltpu.VMEM((EPA,), jnp.int32)]
        + [pltpu.VMEM((K, D), jnp.float32) for _ in range(NRB)]
        + [pltpu.VMEM((K,), jnp.int32) for _ in range(NIB)]
        + [pltpu.VMEM((K,), jnp.float32) for _ in range(NIB)]
        + [pltpu.VMEM_SHARED((NP, D), jnp.float32)]
        + [pltpu.SemaphoreType.DMA for _ in range(2 * NRB + NIB)]
    ),
)
def _agg_kernel(y_hbm, row_hbm, col_hbm, w_hbm, z_hbm, out_hbm, *refs):
    colv = refs[0]
    rows = list(refs[1:1 + NRB])
    ridx = list(refs[1 + NRB:1 + NRB + NIB])
    wbufs = list(refs[1 + NRB + NIB:1 + NRB + 2 * NIB])
    acc = refs[1 + NRB + 2 * NIB]
    sems = refs[2 + NRB + 2 * NIB:]
    gsem = list(sems[:NRB])
    ssem = list(sems[NRB:2 * NRB])
    isem = list(sems[2 * NRB:])

    cid = lax.axis_index("c")
    sid = lax.axis_index("s")
    wid = cid * NS + sid
    r0 = sid * ROWS_PER_TILE
    ebase = wid * EPA
    pltpu.sync_copy(col_hbm.at[pl.ds(ebase, EPA)], colv)
    pltpu.sync_copy(z_hbm.at[pl.ds(r0, ROWS_PER_TILE)],
                    acc.at[pl.ds(r0, ROWS_PER_TILE)])
    plsc.subcore_barrier()

    def _issue_idx(c, ib):
        pltpu.async_copy(row_hbm.at[pl.ds(ebase + c * K, K)], ridx[ib],
                         isem[ib])
        pltpu.async_copy(w_hbm.at[pl.ds(ebase + c * K, K)], wbufs[ib],
                         isem[ib])

    def _wait_idx(c, ib):
        pltpu.make_async_copy(row_hbm.at[pl.ds(ebase + c * K, K)], ridx[ib],
                              isem[ib]).wait()
        pltpu.make_async_copy(w_hbm.at[pl.ds(ebase + c * K, K)], wbufs[ib],
                              isem[ib]).wait()

    def _gather(c, b, ib):
        pltpu.async_copy(y_hbm.at[ridx[ib]], rows[b], gsem[b])

    def _wait_g(c, b, ib):
        pltpu.make_async_copy(y_hbm.at[ridx[ib]], rows[b], gsem[b]).wait()

    def _scatter(c, b):
        pltpu.async_copy(rows[b], acc.at[colv.at[pl.ds(c * K, K)]], ssem[b],
                         add=True)

    def _wait_s(c, b):
        pltpu.make_async_copy(
            rows[b], acc.at[colv.at[pl.ds(c * K, K)]], ssem[b]).wait()

    def _scale(b, ib):
        buf = rows[b]
        wb = wbufs[ib]

        @pl.loop(0, K, step=L)
        def _grp(g):
            wreg = wb[pl.ds(g, L)]
            for j in range(L):
                sp = lax.gather(
                    wreg, jnp.full((L, 1), j, jnp.int32),
                    lax.GatherDimensionNumbers(
                        offset_dims=(), collapsed_slice_dims=(0,),
                        start_index_map=(0,)),
                    slice_sizes=(1,),
                    mode=lax.GatherScatterMode.PROMISE_IN_BOUNDS)
                for cdx in range(D // L):
                    sl = (g + j, pl.ds(cdx * L, L))
                    buf[sl] = buf[sl] * sp

    def _step(c, cc=None):
        b, ib = c % NRB, c % NIB
        _wait_g(cc if cc is not None else c, b, ib)
        _scale(b, ib)
        _scatter(c, b)
        if c >= 1:
            _wait_s_step(c - 1)
        if c + 2 < NCHA:
            _wait_idx_step(c + 2)
            _gather_step(c + 2)
        if c + 4 < NCHA:
            _issue_idx(c + 4, (c + 4) % NIB)

    def _wait_s_step(c):
        _wait_s(c, c % NRB)

    def _wait_idx_step(c):
        _wait_idx(c, c % NIB)

    def _gather_step(c):
        _gather(c, c % NRB, c % NIB)

    for c in range(4):
        _issue_idx(c, c)
    _wait_idx_step(0)
    _gather_step(0)
    _wait_idx_step(1)
    _gather_step(1)

    PRO = 2
    MAIN = (NCHA - PRO - 6) // 6 * 6

    for c in range(PRO):
        _step(c)

    @pl.loop(PRO, PRO + MAIN, step=6)
    def _main(ci):
        for u in range(6):
            c = ci + u
            b, ib = (PRO + u) % NRB, (PRO + u) % NIB
            _wait_g(c, b, ib)
            _scale(b, ib)
            _scatter(c, b)
            _wait_s(c - 1, (b + 2) % NRB)
            _wait_idx(c + 2, (ib + 2) % NIB)
            _gather(c + 2, (b + 2) % NRB, (ib + 2) % NIB)
            _issue_idx(c + 4, (ib + 4) % NIB)

    for c in range(PRO + MAIN, NCHA):
        b, ib = c % NRB, c % NIB
        _wait_g(c, b, ib)
        _scale(b, ib)
        _scatter(c, b)
        if c >= 1:
            _wait_s(c - 1, (c - 1) % NRB)
        if c + 2 < NCHA:
            _wait_idx_step(c + 2)
            _gather_step(c + 2)
        if c + 4 < NCHA:
            _issue_idx(c + 4, (c + 4) % NIB)
    _wait_s(NCHA - 1, (NCHA - 1) % NRB)

    plsc.subcore_barrier()
    pltpu.sync_copy(acc.at[pl.ds(r0, ROWS_PER_TILE)],
                    out_hbm.at[cid, pl.ds(r0, ROWS_PER_TILE)])


def kernel(x, edge_index, edge_weight, W1, W2):
    row = edge_index[0].astype(jnp.int32)
    col = edge_index[1].astype(jnp.int32)
    w = edge_weight.astype(jnp.float32)

    rowp = jnp.pad(row.reshape(NW, EPW), ((0, 0), (0, EPA - EPW))).reshape(EA)
    colp = jnp.pad(col.reshape(NW, EPW), ((0, 0), (0, EPA - EPW))).reshape(EA)
    wp = jnp.pad(w.reshape(NW, EPW), ((0, 0), (0, EPA - EPW))).reshape(EA)
    zD = jnp.zeros((NP, D), jnp.float32)
    xp = jnp.pad(x, ((0, NP - N), (0, 0)))

    degp = _deg_kernel(w, col)
    y1, dis = _lin1(xp, W1, degp)
    p = _agg_kernel(y1, rowp, colp, wp, zD)
    y2 = _mid(p, y1, dis)
    q = _agg_kernel(y2, rowp, colp, wp, zD)
    return _fin(q, y2, dis, W2)[:N]
